# Initial kernel scaffold; baseline (speedup 1.0000x reference)
import jax
import jax.numpy as jnp
from jax.experimental import pallas as pl


def kernel(x, indices):
    raise NotImplementedError("write your pallas kernel here")


# speedup vs baseline: 4.0868x; 4.0868x over previous
import functools

import jax
import jax.numpy as jnp
from jax import lax
from jax.experimental import pallas as pl
from jax.experimental.pallas import tpu as pltpu
from jax.experimental.pallas import tpu_sc as plsc

_B, _C, _H, _W = 4, 96, 192, 192
_BC = _B * _C
_HW = _H * _W
_OUT = 4 * _HW
_NC, _NS = 2, 16
_NW = _NC * _NS
_PPW = _BC // _NW
_NSEG = 3
_SEG = _OUT // _NSEG
_L = 16

_mesh = plsc.VectorSubcoreMesh(core_axis_name="c", subcore_axis_name="s")


@functools.partial(
    pl.kernel,
    out_type=jax.ShapeDtypeStruct((_BC, _OUT), jnp.float32),
    mesh=_mesh,
    compiler_params=pltpu.CompilerParams(needs_layout_passes=False),
    scratch_types=[
        pltpu.VMEM((_HW,), jnp.int32),
        pltpu.VMEM((_HW,), jnp.float32),
        pltpu.VMEM((_SEG,), jnp.float32),
    ],
)
def _scatter_sorted(key_hbm, val_hbm, out_hbm, key_v, val_v, seg_v):
    wid = lax.axis_index("s") * _NC + lax.axis_index("c")

    def plane_body(p_i, carry):
        p = wid * _PPW + p_i
        pltpu.sync_copy(key_hbm.at[p], key_v)
        pltpu.sync_copy(val_hbm.at[p], val_v)
        plane_base = p * _OUT
        for s in range(_NSEG):
            base = plane_base + s * _SEG

            @plsc.parallel_loop(0, _SEG, step=_L, unroll=8)
            def _zero(i):
                seg_v[pl.ds(i, _L)] = jnp.zeros((_L,), jnp.float32)

            def scat(j, c):
                kv = key_v[pl.ds(j * _L, _L)]
                xv = val_v[pl.ds(j * _L, _L)]
                local = kv - base
                m = (local >= 0) & (local < _SEG)
                plsc.store_scatter(seg_v, [jnp.where(m, local, 0)], xv, mask=m)
                return c

            lax.fori_loop(0, _HW // _L, scat, 0, unroll=8)
            pltpu.sync_copy(seg_v, out_hbm.at[p, pl.ds(s * _SEG, _SEG)])
        return carry

    lax.fori_loop(0, _PPW, plane_body, 0)


def kernel(x, indices):
    idx2 = indices.reshape(_BC, _HW)
    keys = (idx2 + (jnp.arange(_BC, dtype=jnp.int32) * _OUT)[:, None]).reshape(-1)
    sk, sv = lax.sort((keys, x.reshape(-1)), dimension=0, num_keys=1,
                      is_stable=False)
    out = _scatter_sorted(sk.reshape(_BC, _HW), sv.reshape(_BC, _HW))
    return out.reshape(_B, _C, 2 * _H, 2 * _W)


# speedup vs baseline: 4.1148x; 1.0068x over previous
import functools

import jax
import jax.numpy as jnp
from jax import lax
from jax.experimental import pallas as pl
from jax.experimental.pallas import tpu as pltpu
from jax.experimental.pallas import tpu_sc as plsc

_B, _C, _H, _W = 4, 96, 192, 192
_BC = _B * _C
_HW = _H * _W
_OUT = 4 * _HW
_NC, _NS = 2, 16
_NW = _NC * _NS
_PPW = _BC // _NW
_NSEG = 3
_SEG = _OUT // _NSEG
_L = 16

_mesh = plsc.VectorSubcoreMesh(core_axis_name="c", subcore_axis_name="s")


@functools.partial(
    pl.kernel,
    out_type=jax.ShapeDtypeStruct((_BC, _OUT), jnp.float32),
    mesh=_mesh,
    compiler_params=pltpu.CompilerParams(needs_layout_passes=False),
    scratch_types=[
        pltpu.VMEM((_HW,), jnp.int32),
        pltpu.VMEM((_HW,), jnp.float32),
        pltpu.VMEM((_SEG,), jnp.float32),
    ],
)
def _scatter_sorted(key_hbm, val_hbm, out_hbm, key_v, val_v, seg_v):
    wid = lax.axis_index("s") * _NC + lax.axis_index("c")

    def plane_body(p_i, carry):
        p = wid * _PPW + p_i
        pltpu.sync_copy(key_hbm.at[pl.ds(p * _HW, _HW)], key_v)
        pltpu.sync_copy(val_hbm.at[pl.ds(p * _HW, _HW)], val_v)
        plane_base = p * _OUT
        for s in range(_NSEG):
            base = plane_base + s * _SEG

            @plsc.parallel_loop(0, _SEG, step=_L, unroll=8)
            def _zero(i):
                seg_v[pl.ds(i, _L)] = jnp.zeros((_L,), jnp.float32)

            def scat(j, c):
                kv = key_v[pl.ds(j * _L, _L)]
                xv = val_v[pl.ds(j * _L, _L)]
                local = kv - base
                m = (local >= 0) & (local < _SEG)
                plsc.store_scatter(seg_v, [jnp.where(m, local, 0)], xv, mask=m)
                return c

            lax.fori_loop(0, _HW // _L, scat, 0, unroll=8)
            pltpu.sync_copy(seg_v, out_hbm.at[p, pl.ds(s * _SEG, _SEG)])
        return carry

    lax.fori_loop(0, _PPW, plane_body, 0)


def kernel(x, indices):
    idx2 = indices.reshape(_BC, _HW)
    keys = (idx2 + (jnp.arange(_BC, dtype=jnp.int32) * _OUT)[:, None]).reshape(-1)
    sk, sv = lax.sort((keys, x.reshape(-1)), dimension=0, num_keys=1,
                      is_stable=False)
    out = _scatter_sorted(sk, sv)
    return out.reshape(_B, _C, 2 * _H, 2 * _W)
